# baseline (speedup 1.0000x reference)
import jax
import jax.numpy as jnp
from jax import lax
from jax.experimental import pallas as pl
from jax.experimental.pallas import tpu as pltpu
from jax.experimental.pallas import tpu_sc as plsc

N = 10000
D = 128
E = 320000
NC = 2
NS = 16
NW = NC * NS
NP = 10240
RPT = NP // NS
ET = E // NW
EP = 10240
NCH = EP // 128
GC = 16
NG = NCH // GC
STEPS = 5
L = 16

_MESH = plsc.VectorSubcoreMesh(core_axis_name="c", subcore_axis_name="s")
f32 = jnp.float32


def _fill(buf, rows, val):
    v = jnp.full((L,), val, f32)

    @pl.loop(0, rows)
    def _f(r):
        for s in range(8):
            buf[r, pl.ds(s * L, L)] = v


def _zero_acc_stripe(acc, zbuf, row0):
    @pl.loop(0, RPT // 16)
    def _z0(i):
        pltpu.sync_copy(zbuf, acc.at[pl.ds(row0 + i * 16, 16)])


def _dump_acc_stripe(acc, p, cid, row0):
    @pl.loop(0, RPT // 128)
    def _dmp(i):
        r0 = row0 + i * 128
        pltpu.sync_copy(acc.at[pl.ds(r0, 128)], p.at[cid, pl.ds(r0, 128)])


def _deg_body(dsts, p, acc, dstb, ones, zbuf, sem0, sem1):
    cid = lax.axis_index("c")
    wid = lax.axis_index("s")
    gid = cid * NS + wid
    row0 = wid * RPT

    _fill(zbuf, 16, 0.0)
    _fill(ones, 128, 1.0)
    _zero_acc_stripe(acc, zbuf, row0)
    plsc.subcore_barrier()

    @pl.loop(0, NG)
    def _deg(g):
        pltpu.sync_copy(dsts.at[gid, pl.ds(g * GC, GC)], dstb)
        sems = (sem0, sem1)
        sd = [None, None]
        for j in range(GC):
            b = j & 1
            if sd[b] is not None:
                sd[b].wait()
            sd[b] = pltpu.async_copy(ones, acc.at[dstb.at[j]], sems[b],
                                     add=True)
        sd[0].wait()
        sd[1].wait()
    plsc.subcore_barrier()
    _dump_acc_stripe(acc, p, cid, row0)


def _tc_init_body(x_ref, p0_ref, p1_ref, h_ref, dinv_ref):
    dv = lax.rsqrt(p0_ref[...] + p1_ref[...])
    dinv_ref[...] = dv
    h_ref[...] = x_ref[...] * dv


def _tc_comb_body(x_ref, dinv_ref, p0_ref, p1_ref, y_ref, h_ref):
    dv = dinv_ref[...]
    yv = 0.5 * ((p0_ref[...] + p1_ref[...]) * dv) + 0.5 * x_ref[...]
    y_ref[...] = yv
    h_ref[...] = yv * dv


def _edge_body(h_hbm, srcs, dsts, p,
               acc, srcb, dstb, msga, msgb, zbuf,
               gsem0, gsem1, ssem0, ssem1):
    cid = lax.axis_index("c")
    wid = lax.axis_index("s")
    gid = cid * NS + wid
    row0 = wid * RPT

    _fill(zbuf, 16, 0.0)
    _zero_acc_stripe(acc, zbuf, row0)
    plsc.subcore_barrier()

    @pl.loop(0, NG)
    def _edge(g):
        pltpu.sync_copy(srcs.at[gid, pl.ds(g * (GC * 128), GC * 128)], srcb)
        pltpu.sync_copy(dsts.at[gid, pl.ds(g * GC, GC)], dstb)
        bufs = (msga, msgb)
        gsems = (gsem0, gsem1)
        ssems = (ssem0, ssem1)
        gd = [None, None]
        sd = [None, None]
        for j in range(GC):
            b = j & 1
            if sd[b] is not None:
                sd[b].wait()
            idx = srcb.at[pl.ds(j * 128, 128)]
            gd[b] = pltpu.async_copy(h_hbm.at[idx], bufs[b], gsems[b])
            if j >= 1:
                pb = (j - 1) & 1
                gd[pb].wait()
                sd[pb] = pltpu.async_copy(
                    bufs[pb], acc.at[dstb.at[j - 1]], ssems[pb], add=True)
        lb = (GC - 1) & 1
        gd[lb].wait()
        sd[lb] = pltpu.async_copy(
            bufs[lb], acc.at[dstb.at[GC - 1]], ssems[lb], add=True)
        sd[0].wait()
        sd[1].wait()
    plsc.subcore_barrier()
    _dump_acc_stripe(acc, p, cid, row0)


_deg_call = pl.kernel(
    _deg_body,
    out_type=jax.ShapeDtypeStruct((NC, NP, D), f32),
    mesh=_MESH,
    scratch_types=[
        pltpu.VMEM_SHARED((NP, D), f32),
        pltpu.VMEM((GC, 128), jnp.int32),
        pltpu.VMEM((128, D), f32),
        pltpu.VMEM((16, D), f32),
        pltpu.SemaphoreType.DMA,
        pltpu.SemaphoreType.DMA,
    ],
)

TB = 512

_init_call = pl.pallas_call(
    _tc_init_body,
    grid=(NP // TB,),
    in_specs=[pl.BlockSpec((TB, D), lambda i: (i, 0))] * 3,
    out_specs=[pl.BlockSpec((TB, D), lambda i: (i, 0))] * 2,
    out_shape=(
        jax.ShapeDtypeStruct((NP, D), f32),
        jax.ShapeDtypeStruct((NP, D), f32),
    ),
)

_edge_call = pl.kernel(
    _edge_body,
    out_type=jax.ShapeDtypeStruct((NC, NP, D), f32),
    mesh=_MESH,
    scratch_types=[
        pltpu.VMEM_SHARED((NP, D), f32),
        pltpu.VMEM((GC * 128,), jnp.int32),
        pltpu.VMEM((GC, 128), jnp.int32),
        pltpu.VMEM((128, D), f32),
        pltpu.VMEM((128, D), f32),
        pltpu.VMEM((16, D), f32),
        pltpu.SemaphoreType.DMA,
        pltpu.SemaphoreType.DMA,
        pltpu.SemaphoreType.DMA,
        pltpu.SemaphoreType.DMA,
    ],
)

_comb_call = pl.pallas_call(
    _tc_comb_body,
    grid=(NP // TB,),
    in_specs=[pl.BlockSpec((TB, D), lambda i: (i, 0))] * 4,
    out_specs=[pl.BlockSpec((TB, D), lambda i: (i, 0))] * 2,
    out_shape=(
        jax.ShapeDtypeStruct((NP, D), f32),
        jax.ShapeDtypeStruct((NP, D), f32),
    ),
)


def kernel(x, edge_index):
    src = edge_index[0].reshape(NW, ET)
    dst = edge_index[1].reshape(NW, ET)
    pad = EP - ET
    src = jnp.pad(src, ((0, 0), (0, pad)))
    dst = jnp.pad(dst, ((0, 0), (0, pad)), constant_values=N)
    dst = dst.reshape(NW, NCH, 128)
    xp = jnp.pad(x, ((0, NP - N), (0, 0)))

    pdeg = _deg_call(dst)
    h, dinv = _init_call(xp, pdeg[0], pdeg[1])
    y = None
    for _ in range(STEPS):
        p = _edge_call(h, src, dst)
        y, h = _comb_call(xp, dinv, p[0], p[1])
    return y[:N]
